# baseline (speedup 1.0000x reference)
import jax
import jax.numpy as jnp
from jax import lax
from jax.experimental import pallas as pl
from jax.experimental.pallas import tpu as pltpu
from jax.experimental.pallas import tpu_sc as plsc

N = 50000
TILES = 16
ROWS_PER_TILE = 3136
NP = TILES * ROWS_PER_TILE
E = 500000
BLK = 384
JS = BLK // 128
NBLK = 84
EDGES_PER_TILE = NBLK * BLK
EP = TILES * EDGES_PER_TILE
EROWS_PER_TILE = EDGES_PER_TILE // 128
D = 128
HW = 64
DUMP_ROW = N

_f32 = jnp.float32
_bf16 = jnp.bfloat16
_i32 = jnp.int32


def _segsum_body(x2, edges, out, eb0, eb1, gi0, gi1, db0, db1,
                 rw0, rw1, zbuf, acc, es0, es1, gs0, gs1, ss0, ss1):
    c = lax.axis_index("c")
    s = lax.axis_index("s")
    ebuf = (eb0, eb1)
    gidx = (gi0, gi1)
    dstb = (db0, db1)
    rows = (rw0, rw1)
    esem = (es0, es1)
    gsem = (gs0, gs1)
    ssem = (ss0, ss1)

    zv = jnp.zeros((32,), _bf16)

    def _zb(i, carry):
        zbuf[i, pl.ds(0, 32)] = zv
        zbuf[i, pl.ds(32, 32)] = zv
        return carry

    lax.fori_loop(0, 64, _zb, 0)

    row0 = s * ROWS_PER_TILE
    erow0 = s * EROWS_PER_TILE

    if True:
        base = c

        def _fire_idx(b, par):
            return pltpu.async_copy(edges.at[pl.ds(erow0 + b * JS, JS)],
                                    ebuf[par], esem[par])

        def _fire_gather(par):
            for j in range(JS):
                for v in range(8):
                    sl = pl.ds(v * 16, 16)
                    gidx[par][j, sl] = ebuf[par][j, 0, sl] * 2 + base
                    dstb[par][j, sl] = ebuf[par][j, 1, sl]
            return [pltpu.async_copy(x2.at[gidx[par].at[j]],
                                     rows[par].at[j], gsem[par])
                    for j in range(JS)]

        def _scatter_wait(par):
            for j in range(JS):
                pltpu.make_async_copy(x2.at[gidx[par].at[j]],
                                      rows[par].at[j], gsem[par]).wait()
            hs = [pltpu.async_copy(rows[par].at[j], acc.at[dstb[par].at[j]],
                                   ssem[par], add=True)
                  for j in range(JS)]
            for h in hs:
                h.wait()

        h_idx0 = _fire_idx(0, 0)
        for k in range(ROWS_PER_TILE // 64):
            pltpu.sync_copy(zbuf, acc.at[pl.ds(row0 + k * 64, 64)])
        plsc.subcore_barrier()

        h_idx0.wait()
        _fire_gather(0)
        _fire_idx(1, 1).wait()

        def _pair(p, carry):
            b = p * 2
            _fire_gather(1)
            h2 = _fire_idx(b + 2, 0)
            _scatter_wait(0)
            h2.wait()
            _fire_gather(0)

            @pl.when(b + 3 < NBLK)
            def _():
                _fire_idx(b + 3, 1).wait()

            _scatter_wait(1)
            return carry

        lax.fori_loop(0, NBLK // 2 - 1, _pair, 0)
        _fire_gather(1)
        _scatter_wait(0)
        _scatter_wait(1)
        plsc.subcore_barrier()

        pltpu.sync_copy(acc.at[pl.ds(row0, ROWS_PER_TILE)],
                        out.at[c, pl.ds(row0, ROWS_PER_TILE)])
        plsc.subcore_barrier()


@jax.jit
def _segsum(h_src, edges):
    x2 = h_src.reshape(NP * 2, HW)
    mesh = plsc.VectorSubcoreMesh(core_axis_name="c", subcore_axis_name="s")
    f = pl.kernel(
        _segsum_body,
        out_type=jax.ShapeDtypeStruct((2, NP, HW), _bf16),
        mesh=mesh,
        scratch_types=(
            [pltpu.VMEM((JS, 2, 128), _i32)] * 2
            + [pltpu.VMEM((JS, 128), _i32)] * 2
            + [pltpu.VMEM((JS, 128), _i32)] * 2
            + [pltpu.VMEM((JS, 128, HW), _bf16)] * 2
            + [pltpu.VMEM((64, HW), _bf16)]
            + [pltpu.VMEM_SHARED((NP, HW), _bf16)]
            + [pltpu.SemaphoreType.DMA] * 6
        ),
        compiler_params=pltpu.CompilerParams(use_tc_tiling_on_sc=False),
    )
    return f(x2, edges)


_GRID = NP // ROWS_PER_TILE
_CT = (((1,), (1,)), ((), ()))


def _init_body(xu_ref, xi_ref, w_ref, b_ref, hu_ref, hi_ref):
    for t in range(2):
        x = (xu_ref, xi_ref)[t][...]
        y = lax.dot_general(x, w_ref[t], _CT, preferred_element_type=_f32)
        (hu_ref, hi_ref)[t][...] = jnp.maximum(y + b_ref[t], 0.0).astype(_bf16)


def _init(x_user, x_item, w_st, b_st):
    return pl.pallas_call(
        _init_body,
        grid=(_GRID,),
        in_specs=[
            pl.BlockSpec((ROWS_PER_TILE, D), lambda i: (i, 0)),
            pl.BlockSpec((ROWS_PER_TILE, D), lambda i: (i, 0)),
            pl.BlockSpec((2, D, D), lambda i: (0, 0, 0)),
            pl.BlockSpec((2, 1, D), lambda i: (0, 0, 0)),
        ],
        out_specs=[
            pl.BlockSpec((ROWS_PER_TILE, D), lambda i: (i, 0)),
            pl.BlockSpec((ROWS_PER_TILE, D), lambda i: (i, 0)),
        ],
        out_shape=[
            jax.ShapeDtypeStruct((NP, D), _bf16),
            jax.ShapeDtypeStruct((NP, D), _bf16),
        ],
    )(x_user, x_item, w_st, b_st)


def _combine_body(a_ref, h_ref, wrel_ref, b_ref, wroot_ref, o_ref):
    y = lax.dot_general(h_ref[...], wroot_ref[...], _CT,
                        preferred_element_type=_f32)
    for h in range(2):
        wh = wrel_ref[:, h * HW:(h + 1) * HW]
        y = y + lax.dot_general(a_ref[h], wh, _CT,
                                preferred_element_type=_f32)
    o_ref[...] = jnp.maximum(y + b_ref[...], 0.0).astype(_bf16)


def _combine(a, h, wrel, b2, wroot):
    return pl.pallas_call(
        _combine_body,
        grid=(_GRID,),
        in_specs=[
            pl.BlockSpec((2, ROWS_PER_TILE, HW), lambda i: (0, i, 0)),
            pl.BlockSpec((ROWS_PER_TILE, D), lambda i: (i, 0)),
            pl.BlockSpec((D, D), lambda i: (0, 0)),
            pl.BlockSpec((1, D), lambda i: (0, 0)),
            pl.BlockSpec((D, D), lambda i: (0, 0)),
        ],
        out_specs=pl.BlockSpec((ROWS_PER_TILE, D), lambda i: (i, 0)),
        out_shape=jax.ShapeDtypeStruct((NP, D), _bf16),
    )(a, h, wrel, b2, wroot)


def _final_body(hu_ref, hi_ref, w_ref, b_ref, ou_ref, oi_ref):
    for t in range(2):
        y = lax.dot_general((hu_ref, hi_ref)[t][...], w_ref[...], _CT,
                            preferred_element_type=_f32)
        (ou_ref, oi_ref)[t][...] = y + b_ref[...]


def _final(hu, hi, w, b2):
    return pl.pallas_call(
        _final_body,
        grid=(_GRID,),
        in_specs=[
            pl.BlockSpec((ROWS_PER_TILE, D), lambda i: (i, 0)),
            pl.BlockSpec((ROWS_PER_TILE, D), lambda i: (i, 0)),
            pl.BlockSpec((D, D), lambda i: (0, 0)),
            pl.BlockSpec((1, D), lambda i: (0, 0)),
        ],
        out_specs=[
            pl.BlockSpec((ROWS_PER_TILE, D), lambda i: (i, 0)),
            pl.BlockSpec((ROWS_PER_TILE, D), lambda i: (i, 0)),
        ],
        out_shape=[
            jax.ShapeDtypeStruct((N, D), _f32),
            jax.ShapeDtypeStruct((N, D), _f32),
        ],
    )(hu, hi, w, b2)


def _prep_edges(ei):
    src = jnp.concatenate([ei[0], jnp.zeros((EP - E,), _i32)])
    dst = jnp.concatenate([ei[1], jnp.full((EP - E,), DUMP_ROW, _i32)])
    return jnp.stack([src.reshape(EP // 128, 128),
                      dst.reshape(EP // 128, 128)], axis=1)


def kernel(x_user, x_item, ei_user_to_item, ei_item_to_user, W_lin_user,
           b_lin_user, W_lin_item, b_lin_item, W_rel_u2i_0, b_rel_u2i_0,
           W_root_u2i_0, W_rel_i2u_0, b_rel_i2u_0, W_root_i2u_0, W_rel_u2i_1,
           b_rel_u2i_1, W_root_u2i_1, W_rel_i2u_1, b_rel_i2u_1, W_root_i2u_1,
           W_rel_u2i_2, b_rel_u2i_2, W_root_u2i_2, W_rel_i2u_2, b_rel_i2u_2,
           W_root_i2u_2, W_out, b_out):
    ed_iu = _prep_edges(ei_item_to_user)
    ed_ui = _prep_edges(ei_user_to_item)

    h_u, h_i = _init(x_user, x_item,
                     jnp.stack([W_lin_user, W_lin_item]),
                     jnp.stack([b_lin_user.reshape(1, D),
                                b_lin_item.reshape(1, D)]))

    rel_u2i = (W_rel_u2i_0, W_rel_u2i_1, W_rel_u2i_2)
    brel_u2i = (b_rel_u2i_0, b_rel_u2i_1, b_rel_u2i_2)
    root_u2i = (W_root_u2i_0, W_root_u2i_1, W_root_u2i_2)
    rel_i2u = (W_rel_i2u_0, W_rel_i2u_1, W_rel_i2u_2)
    brel_i2u = (b_rel_i2u_0, b_rel_i2u_1, b_rel_i2u_2)
    root_i2u = (W_root_i2u_0, W_root_i2u_1, W_root_i2u_2)

    for l in range(3):
        agg_i = _segsum(h_u, ed_ui)
        agg_u = _segsum(h_i, ed_iu)
        new_i = _combine(agg_i, h_i, rel_u2i[l],
                         brel_u2i[l].reshape(1, D), root_u2i[l])
        new_u = _combine(agg_u, h_u, rel_i2u[l],
                         brel_i2u[l].reshape(1, D), root_i2u[l])
        h_u, h_i = new_u, new_i

    y_u, y_i = _final(h_u, h_i, W_out, b_out.reshape(1, D))
    return (y_u, y_i)


# speedup vs baseline: 1.3977x; 1.3977x over previous
import jax
import jax.numpy as jnp
from jax import lax
from jax.experimental import pallas as pl
from jax.experimental.pallas import tpu as pltpu
from jax.experimental.pallas import tpu_sc as plsc

N = 50000
TILES = 16
ROWS_PER_TILE = 3136
NP = TILES * ROWS_PER_TILE
E = 500000
BLK = 256
JS = BLK // 128
NBLK = 124
EDGES_PER_TILE = NBLK * BLK
EP = TILES * EDGES_PER_TILE
EROWS_PER_TILE = EDGES_PER_TILE // 128
D = 128
HW = 64
DUMP_ROW = N

_f32 = jnp.float32
_bf16 = jnp.bfloat16
_i32 = jnp.int32


def _segsum_body(x2, edges, out, eb0, eb1, gi0, gi1, db0, db1,
                 rw0, rw1, zbuf, acc, es0, es1, gs0, gs1, ss0, ss1):
    c = lax.axis_index("c")
    s = lax.axis_index("s")
    ebuf = (eb0, eb1)
    gidx = (gi0, gi1)
    dstb = (db0, db1)
    rows = (rw0, rw1)
    esem = (es0, es1)
    gsem = (gs0, gs1)
    ssem = (ss0, ss1)

    zv = jnp.zeros((32,), _bf16)

    def _zb(i, carry):
        zbuf[i, pl.ds(0, 32)] = zv
        zbuf[i, pl.ds(32, 32)] = zv
        return carry

    lax.fori_loop(0, 64, _zb, 0)

    row0 = s * ROWS_PER_TILE
    erow0 = s * EROWS_PER_TILE

    if True:
        base = c

        def _fire_idx(b, par):
            return pltpu.async_copy(edges.at[pl.ds(erow0 + b * JS, JS)],
                                    ebuf[par], esem[par])

        def _fire_gather(par):
            for j in range(JS):
                for v in range(8):
                    sl = pl.ds(v * 16, 16)
                    gidx[par][j, sl] = ebuf[par][j, 0, sl] * 2 + base
                    dstb[par][j, sl] = ebuf[par][j, 1, sl]
            return [pltpu.async_copy(x2.at[gidx[par].at[j]],
                                     rows[par].at[j], gsem[par])
                    for j in range(JS)]

        def _scatter_wait(par):
            for j in range(JS):
                pltpu.make_async_copy(x2.at[gidx[par].at[j]],
                                      rows[par].at[j], gsem[par]).wait()
            hs = [pltpu.async_copy(rows[par].at[j], acc.at[dstb[par].at[j]],
                                   ssem[par], add=True)
                  for j in range(JS)]
            for h in hs:
                h.wait()

        h_idx0 = _fire_idx(0, 0)
        for k in range(ROWS_PER_TILE // 64):
            pltpu.sync_copy(zbuf, acc.at[pl.ds(row0 + k * 64, 64)])
        plsc.subcore_barrier()

        h_idx0.wait()
        _fire_gather(0)
        _fire_idx(1, 1).wait()

        def _pair(p, carry):
            b = p * 2
            _fire_gather(1)

            @pl.when(b + 3 < NBLK)
            def _():
                _fire_idx(b + 3, 1)

            h2 = _fire_idx(b + 2, 0)
            _scatter_wait(0)
            h2.wait()
            _fire_gather(0)
            _scatter_wait(1)

            @pl.when(b + 3 < NBLK)
            def _():
                pltpu.make_async_copy(edges.at[pl.ds(erow0, JS)],
                                      ebuf[1], esem[1]).wait()
            return carry

        lax.fori_loop(0, NBLK // 2 - 1, _pair, 0)
        _fire_gather(1)
        _scatter_wait(0)
        _scatter_wait(1)
        plsc.subcore_barrier()

        pltpu.sync_copy(acc.at[pl.ds(row0, ROWS_PER_TILE)],
                        out.at[c, pl.ds(row0, ROWS_PER_TILE)])
        plsc.subcore_barrier()


@jax.jit
def _segsum(h_src, edges):
    x2 = h_src.reshape(NP * 2, HW)
    mesh = plsc.VectorSubcoreMesh(core_axis_name="c", subcore_axis_name="s")
    f = pl.kernel(
        _segsum_body,
        out_type=jax.ShapeDtypeStruct((2, NP, HW), _bf16),
        mesh=mesh,
        scratch_types=(
            [pltpu.VMEM((JS, 2, 128), _i32)] * 2
            + [pltpu.VMEM((JS, 128), _i32)] * 2
            + [pltpu.VMEM((JS, 128), _i32)] * 2
            + [pltpu.VMEM((JS, 128, HW), _bf16)] * 2
            + [pltpu.VMEM((64, HW), _bf16)]
            + [pltpu.VMEM_SHARED((NP, HW), _bf16)]
            + [pltpu.SemaphoreType.DMA] * 6
        ),
        compiler_params=pltpu.CompilerParams(use_tc_tiling_on_sc=False),
    )
    return f(x2, edges)


_GRID = NP // ROWS_PER_TILE
_CT = (((1,), (1,)), ((), ()))


def _init_body(xu_ref, xi_ref, w_ref, b_ref, hu_ref, hi_ref):
    for t in range(2):
        x = (xu_ref, xi_ref)[t][...]
        y = lax.dot_general(x, w_ref[t], _CT, preferred_element_type=_f32)
        (hu_ref, hi_ref)[t][...] = jnp.maximum(y + b_ref[t], 0.0).astype(_bf16)


def _init(x_user, x_item, w_st, b_st):
    return pl.pallas_call(
        _init_body,
        grid=(_GRID,),
        in_specs=[
            pl.BlockSpec((ROWS_PER_TILE, D), lambda i: (i, 0)),
            pl.BlockSpec((ROWS_PER_TILE, D), lambda i: (i, 0)),
            pl.BlockSpec((2, D, D), lambda i: (0, 0, 0)),
            pl.BlockSpec((2, 1, D), lambda i: (0, 0, 0)),
        ],
        out_specs=[
            pl.BlockSpec((ROWS_PER_TILE, D), lambda i: (i, 0)),
            pl.BlockSpec((ROWS_PER_TILE, D), lambda i: (i, 0)),
        ],
        out_shape=[
            jax.ShapeDtypeStruct((NP, D), _bf16),
            jax.ShapeDtypeStruct((NP, D), _bf16),
        ],
    )(x_user, x_item, w_st, b_st)


def _combine_body(a_ref, h_ref, wrel_ref, b_ref, wroot_ref, o_ref):
    y = lax.dot_general(h_ref[...], wroot_ref[...], _CT,
                        preferred_element_type=_f32)
    for h in range(2):
        wh = wrel_ref[:, h * HW:(h + 1) * HW]
        y = y + lax.dot_general(a_ref[h], wh, _CT,
                                preferred_element_type=_f32)
    o_ref[...] = jnp.maximum(y + b_ref[...], 0.0).astype(_bf16)


def _combine(a, h, wrel, b2, wroot):
    return pl.pallas_call(
        _combine_body,
        grid=(_GRID,),
        in_specs=[
            pl.BlockSpec((2, ROWS_PER_TILE, HW), lambda i: (0, i, 0)),
            pl.BlockSpec((ROWS_PER_TILE, D), lambda i: (i, 0)),
            pl.BlockSpec((D, D), lambda i: (0, 0)),
            pl.BlockSpec((1, D), lambda i: (0, 0)),
            pl.BlockSpec((D, D), lambda i: (0, 0)),
        ],
        out_specs=pl.BlockSpec((ROWS_PER_TILE, D), lambda i: (i, 0)),
        out_shape=jax.ShapeDtypeStruct((NP, D), _bf16),
    )(a, h, wrel, b2, wroot)


def _final_body(hu_ref, hi_ref, w_ref, b_ref, ou_ref, oi_ref):
    for t in range(2):
        y = lax.dot_general((hu_ref, hi_ref)[t][...], w_ref[...], _CT,
                            preferred_element_type=_f32)
        (ou_ref, oi_ref)[t][...] = y + b_ref[...]


def _final(hu, hi, w, b2):
    return pl.pallas_call(
        _final_body,
        grid=(_GRID,),
        in_specs=[
            pl.BlockSpec((ROWS_PER_TILE, D), lambda i: (i, 0)),
            pl.BlockSpec((ROWS_PER_TILE, D), lambda i: (i, 0)),
            pl.BlockSpec((D, D), lambda i: (0, 0)),
            pl.BlockSpec((1, D), lambda i: (0, 0)),
        ],
        out_specs=[
            pl.BlockSpec((ROWS_PER_TILE, D), lambda i: (i, 0)),
            pl.BlockSpec((ROWS_PER_TILE, D), lambda i: (i, 0)),
        ],
        out_shape=[
            jax.ShapeDtypeStruct((N, D), _f32),
            jax.ShapeDtypeStruct((N, D), _f32),
        ],
    )(hu, hi, w, b2)


def _prep_edges(ei):
    src = jnp.concatenate([ei[0], jnp.zeros((EP - E,), _i32)])
    dst = jnp.concatenate([ei[1], jnp.full((EP - E,), DUMP_ROW, _i32)])
    return jnp.stack([src.reshape(EP // 128, 128),
                      dst.reshape(EP // 128, 128)], axis=1)


def kernel(x_user, x_item, ei_user_to_item, ei_item_to_user, W_lin_user,
           b_lin_user, W_lin_item, b_lin_item, W_rel_u2i_0, b_rel_u2i_0,
           W_root_u2i_0, W_rel_i2u_0, b_rel_i2u_0, W_root_i2u_0, W_rel_u2i_1,
           b_rel_u2i_1, W_root_u2i_1, W_rel_i2u_1, b_rel_i2u_1, W_root_i2u_1,
           W_rel_u2i_2, b_rel_u2i_2, W_root_u2i_2, W_rel_i2u_2, b_rel_i2u_2,
           W_root_i2u_2, W_out, b_out):
    ed_iu = _prep_edges(ei_item_to_user)
    ed_ui = _prep_edges(ei_user_to_item)

    h_u, h_i = _init(x_user, x_item,
                     jnp.stack([W_lin_user, W_lin_item]),
                     jnp.stack([b_lin_user.reshape(1, D),
                                b_lin_item.reshape(1, D)]))

    rel_u2i = (W_rel_u2i_0, W_rel_u2i_1, W_rel_u2i_2)
    brel_u2i = (b_rel_u2i_0, b_rel_u2i_1, b_rel_u2i_2)
    root_u2i = (W_root_u2i_0, W_root_u2i_1, W_root_u2i_2)
    rel_i2u = (W_rel_i2u_0, W_rel_i2u_1, W_rel_i2u_2)
    brel_i2u = (b_rel_i2u_0, b_rel_i2u_1, b_rel_i2u_2)
    root_i2u = (W_root_i2u_0, W_root_i2u_1, W_root_i2u_2)

    for l in range(3):
        agg_i = _segsum(h_u, ed_ui)
        agg_u = _segsum(h_i, ed_iu)
        new_i = _combine(agg_i, h_i, rel_u2i[l],
                         brel_u2i[l].reshape(1, D), root_u2i[l])
        new_u = _combine(agg_u, h_u, rel_i2u[l],
                         brel_i2u[l].reshape(1, D), root_i2u[l])
        h_u, h_i = new_u, new_i

    y_u, y_i = _final(h_u, h_i, W_out, b_out.reshape(1, D))
    return (y_u, y_i)


# speedup vs baseline: 1.4092x; 1.0082x over previous
import jax
import jax.numpy as jnp
from jax import lax
from jax.experimental import pallas as pl
from jax.experimental.pallas import tpu as pltpu
from jax.experimental.pallas import tpu_sc as plsc

N = 50000
TILES = 16
ROWS_PER_TILE = 3136
NP = TILES * ROWS_PER_TILE
E = 500000
BLK = 256
JS = BLK // 128
NBLK = 124
EDGES_PER_TILE = NBLK * BLK
EP = TILES * EDGES_PER_TILE
EROWS_PER_TILE = EDGES_PER_TILE // 128
D = 128
HW = 64
DUMP_ROW = N

_f32 = jnp.float32
_bf16 = jnp.bfloat16
_i32 = jnp.int32


def _segsum_body(x2, edges, out, eb0, eb1, gi0, gi1, db0, db1,
                 rw0, rw1, zbuf, acc, es0, es1, gs0, gs1, ss0, ss1):
    c = lax.axis_index("c")
    s = lax.axis_index("s")
    ebuf = (eb0, eb1)
    gidx = (gi0, gi1)
    dstb = (db0, db1)
    rows = (rw0, rw1)
    esem = (es0, es1)
    gsem = (gs0, gs1)
    ssem = (ss0, ss1)

    zv = jnp.zeros((32,), _bf16)

    def _zb(i, carry):
        zbuf[i, pl.ds(0, 32)] = zv
        zbuf[i, pl.ds(32, 32)] = zv
        return carry

    lax.fori_loop(0, 64, _zb, 0)

    row0 = s * ROWS_PER_TILE
    erow0 = s * EROWS_PER_TILE

    if True:
        base = c

        def _fire_idx(b, par):
            return pltpu.async_copy(
                edges.at[pl.ds(2 * (erow0 + b * JS), 2 * JS)],
                ebuf[par], esem[par])

        def _fire_gather(par):
            for j in range(JS):
                for v in range(8):
                    sl = pl.ds(v * 16, 16)
                    gidx[par][j, sl] = ebuf[par][2 * j, sl] * 2 + base
                    dstb[par][j, sl] = ebuf[par][2 * j + 1, sl]
            return [pltpu.async_copy(x2.at[gidx[par].at[j]],
                                     rows[par].at[j], gsem[par])
                    for j in range(JS)]

        def _scatter_wait(par):
            for j in range(JS):
                pltpu.make_async_copy(x2.at[gidx[par].at[j]],
                                      rows[par].at[j], gsem[par]).wait()
            hs = [pltpu.async_copy(rows[par].at[j], acc.at[dstb[par].at[j]],
                                   ssem[par], add=True)
                  for j in range(JS)]
            for h in hs:
                h.wait()

        h_idx0 = _fire_idx(0, 0)
        for k in range(ROWS_PER_TILE // 64):
            pltpu.sync_copy(zbuf, acc.at[pl.ds(row0 + k * 64, 64)])
        plsc.subcore_barrier()

        h_idx0.wait()
        _fire_gather(0)
        _fire_idx(1, 1).wait()

        def _pair(p, carry):
            b = p * 2
            _fire_gather(1)

            @pl.when(b + 3 < NBLK)
            def _():
                _fire_idx(b + 3, 1)

            h2 = _fire_idx(b + 2, 0)
            _scatter_wait(0)
            h2.wait()
            _fire_gather(0)
            _scatter_wait(1)

            @pl.when(b + 3 < NBLK)
            def _():
                pltpu.make_async_copy(edges.at[pl.ds(2 * erow0, 2 * JS)],
                                      ebuf[1], esem[1]).wait()
            return carry

        lax.fori_loop(0, NBLK // 2 - 1, _pair, 0)
        _fire_gather(1)
        _scatter_wait(0)
        _scatter_wait(1)
        plsc.subcore_barrier()

        pltpu.sync_copy(acc.at[pl.ds(row0, ROWS_PER_TILE)],
                        out.at[pl.ds(row0, ROWS_PER_TILE),
                               pl.ds(c * HW, HW)])
        plsc.subcore_barrier()


@jax.jit
def _segsum(h_src, edges):
    x2 = h_src.reshape(NP * 2, HW)
    mesh = plsc.VectorSubcoreMesh(core_axis_name="c", subcore_axis_name="s")
    f = pl.kernel(
        _segsum_body,
        out_type=jax.ShapeDtypeStruct((NP, D), _bf16),
        mesh=mesh,
        scratch_types=(
            [pltpu.VMEM((2 * JS, 128), _i32)] * 2
            + [pltpu.VMEM((JS, 128), _i32)] * 2
            + [pltpu.VMEM((JS, 128), _i32)] * 2
            + [pltpu.VMEM((JS, 128, HW), _bf16)] * 2
            + [pltpu.VMEM((64, HW), _bf16)]
            + [pltpu.VMEM_SHARED((NP, HW), _bf16)]
            + [pltpu.SemaphoreType.DMA] * 6
        ),
        compiler_params=pltpu.CompilerParams(use_tc_tiling_on_sc=False),
    )
    return f(x2, edges)


_GRID = NP // ROWS_PER_TILE
_CT = (((1,), (1,)), ((), ()))


def _init_body(xu_ref, xi_ref, w_ref, b_ref, hu_ref, hi_ref):
    for t in range(2):
        x = (xu_ref, xi_ref)[t][...]
        y = lax.dot_general(x, w_ref[t], _CT, preferred_element_type=_f32)
        (hu_ref, hi_ref)[t][...] = jnp.maximum(y + b_ref[t], 0.0).astype(_bf16)


def _init(x_user, x_item, w_st, b_st):
    return pl.pallas_call(
        _init_body,
        grid=(_GRID,),
        in_specs=[
            pl.BlockSpec((ROWS_PER_TILE, D), lambda i: (i, 0)),
            pl.BlockSpec((ROWS_PER_TILE, D), lambda i: (i, 0)),
            pl.BlockSpec((2, D, D), lambda i: (0, 0, 0)),
            pl.BlockSpec((2, 1, D), lambda i: (0, 0, 0)),
        ],
        out_specs=[
            pl.BlockSpec((ROWS_PER_TILE, D), lambda i: (i, 0)),
            pl.BlockSpec((ROWS_PER_TILE, D), lambda i: (i, 0)),
        ],
        out_shape=[
            jax.ShapeDtypeStruct((NP, D), _bf16),
            jax.ShapeDtypeStruct((NP, D), _bf16),
        ],
    )(x_user, x_item, w_st, b_st)


def _combine_body(a_ref, h_ref, wrel_ref, b_ref, wroot_ref, o_ref):
    y = lax.dot_general(h_ref[...], wroot_ref[...], _CT,
                        preferred_element_type=_f32)
    y = y + lax.dot_general(a_ref[...], wrel_ref[...], _CT,
                            preferred_element_type=_f32)
    o_ref[...] = jnp.maximum(y + b_ref[...], 0.0).astype(_bf16)


def _combine(a, h, wrel, b2, wroot):
    return pl.pallas_call(
        _combine_body,
        grid=(_GRID,),
        in_specs=[
            pl.BlockSpec((ROWS_PER_TILE, D), lambda i: (i, 0)),
            pl.BlockSpec((ROWS_PER_TILE, D), lambda i: (i, 0)),
            pl.BlockSpec((D, D), lambda i: (0, 0)),
            pl.BlockSpec((1, D), lambda i: (0, 0)),
            pl.BlockSpec((D, D), lambda i: (0, 0)),
        ],
        out_specs=pl.BlockSpec((ROWS_PER_TILE, D), lambda i: (i, 0)),
        out_shape=jax.ShapeDtypeStruct((NP, D), _bf16),
    )(a, h, wrel, b2, wroot)


def _final_body(hu_ref, hi_ref, w_ref, b_ref, ou_ref, oi_ref):
    for t in range(2):
        y = lax.dot_general((hu_ref, hi_ref)[t][...], w_ref[...], _CT,
                            preferred_element_type=_f32)
        (ou_ref, oi_ref)[t][...] = y + b_ref[...]


def _final(hu, hi, w, b2):
    return pl.pallas_call(
        _final_body,
        grid=(_GRID,),
        in_specs=[
            pl.BlockSpec((ROWS_PER_TILE, D), lambda i: (i, 0)),
            pl.BlockSpec((ROWS_PER_TILE, D), lambda i: (i, 0)),
            pl.BlockSpec((D, D), lambda i: (0, 0)),
            pl.BlockSpec((1, D), lambda i: (0, 0)),
        ],
        out_specs=[
            pl.BlockSpec((ROWS_PER_TILE, D), lambda i: (i, 0)),
            pl.BlockSpec((ROWS_PER_TILE, D), lambda i: (i, 0)),
        ],
        out_shape=[
            jax.ShapeDtypeStruct((N, D), _f32),
            jax.ShapeDtypeStruct((N, D), _f32),
        ],
    )(hu, hi, w, b2)


def _prep_edges(ei):
    src = jnp.concatenate([ei[0], jnp.zeros((EP - E,), _i32)])
    dst = jnp.concatenate([ei[1], jnp.full((EP - E,), DUMP_ROW, _i32)])
    return jnp.stack([src.reshape(EP // 128, 128),
                      dst.reshape(EP // 128, 128)],
                     axis=1).reshape(2 * EP // 128, 128)


def kernel(x_user, x_item, ei_user_to_item, ei_item_to_user, W_lin_user,
           b_lin_user, W_lin_item, b_lin_item, W_rel_u2i_0, b_rel_u2i_0,
           W_root_u2i_0, W_rel_i2u_0, b_rel_i2u_0, W_root_i2u_0, W_rel_u2i_1,
           b_rel_u2i_1, W_root_u2i_1, W_rel_i2u_1, b_rel_i2u_1, W_root_i2u_1,
           W_rel_u2i_2, b_rel_u2i_2, W_root_u2i_2, W_rel_i2u_2, b_rel_i2u_2,
           W_root_i2u_2, W_out, b_out):
    ed_iu = _prep_edges(ei_item_to_user)
    ed_ui = _prep_edges(ei_user_to_item)

    h_u, h_i = _init(x_user, x_item,
                     jnp.stack([W_lin_user, W_lin_item]),
                     jnp.stack([b_lin_user.reshape(1, D),
                                b_lin_item.reshape(1, D)]))

    rel_u2i = (W_rel_u2i_0, W_rel_u2i_1, W_rel_u2i_2)
    brel_u2i = (b_rel_u2i_0, b_rel_u2i_1, b_rel_u2i_2)
    root_u2i = (W_root_u2i_0, W_root_u2i_1, W_root_u2i_2)
    rel_i2u = (W_rel_i2u_0, W_rel_i2u_1, W_rel_i2u_2)
    brel_i2u = (b_rel_i2u_0, b_rel_i2u_1, b_rel_i2u_2)
    root_i2u = (W_root_i2u_0, W_root_i2u_1, W_root_i2u_2)

    for l in range(3):
        agg_i = _segsum(h_u, ed_ui)
        agg_u = _segsum(h_i, ed_iu)
        new_i = _combine(agg_i, h_i, rel_u2i[l],
                         brel_u2i[l].reshape(1, D), root_u2i[l])
        new_u = _combine(agg_u, h_u, rel_i2u[l],
                         brel_i2u[l].reshape(1, D), root_i2u[l])
        h_u, h_i = new_u, new_i

    y_u, y_i = _final(h_u, h_i, W_out, b_out.reshape(1, D))
    return (y_u, y_i)
